# baseline (speedup 1.0000x reference)
import jax
import jax.numpy as jnp
from jax import lax
from jax.experimental import pallas as pl
from jax.experimental.pallas import tpu as pltpu

_MAX_MASK_RATIO = 0.1
_T_BLK = 512


def _mask_params(B, D):
    max_mask_len = int(D * _MAX_MASK_RATIO)
    key = jax.random.key(42)
    k1, k2 = jax.random.split(key)
    mask_len = jax.random.randint(k1, (B,), 1, max_mask_len + 1)
    mask_start = jax.random.randint(k2, (B,), 0, D - max_mask_len + 1)
    return mask_start.astype(jnp.int32), (mask_start + mask_len).astype(jnp.int32)


def _body(starts_ref, ends_ref, x_ref, o_ref):
    b = pl.program_id(0)
    s = starts_ref[b]
    e = ends_ref[b]
    col = lax.broadcasted_iota(jnp.int32, (_T_BLK, x_ref.shape[-1]), 1)
    keep = (col < s) | (col >= e)
    o_ref[0] = jnp.where(keep, x_ref[0], 0.0)


def kernel(mean):
    B, T, D = mean.shape
    starts, ends = _mask_params(B, D)
    grid = (B, T // _T_BLK)
    return pl.pallas_call(
        _body,
        grid=grid,
        in_specs=[
            pl.BlockSpec(memory_space=pltpu.SMEM),
            pl.BlockSpec(memory_space=pltpu.SMEM),
            pl.BlockSpec((1, _T_BLK, D), lambda b, t: (b, t, 0)),
        ],
        out_specs=pl.BlockSpec((1, _T_BLK, D), lambda b, t: (b, t, 0)),
        out_shape=jax.ShapeDtypeStruct((B, T, D), mean.dtype),
    )(starts, ends, mean)
